# Initial kernel scaffold; baseline (speedup 1.0000x reference)
import functools

import jax
import jax.numpy as jnp
from jax import lax
from jax.experimental import pallas as pl
from jax.experimental.pallas import tpu as pltpu
from jax.experimental.pallas import tpu_sc as plsc

_EMB = 64
_N = 4096 * 26
_NC = 2
_NS = 16
_NW = _NC * _NS
_BPW = _N // _NW
_NBUF = 2
_CHUNK = 832
_NCHUNK = _BPW // _CHUNK

_mesh = plsc.VectorSubcoreMesh(core_axis_name="c", subcore_axis_name="s")


@functools.partial(
    pl.kernel,
    mesh=_mesh,
    out_type=jax.ShapeDtypeStruct((_N, _EMB), jnp.float32),
    scratch_types=[
        pltpu.VMEM((_NCHUNK, _CHUNK), jnp.int32),
        pltpu.VMEM((_CHUNK, _EMB), jnp.float32),
        pltpu.VMEM((_CHUNK, _EMB), jnp.float32),
        pltpu.SemaphoreType.DMA,
        pltpu.SemaphoreType.DMA,
        pltpu.SemaphoreType.DMA,
        pltpu.SemaphoreType.DMA,
    ],
)
def _emb_lookup(idx_hbm, table_hbm, out_hbm, idx_v, rows0, rows1,
                gsem0, gsem1, osem0, osem1):
    wid = lax.axis_index("s") * _NC + lax.axis_index("c")
    base = wid * _BPW

    pltpu.sync_copy(idx_hbm.at[wid], idx_v)

    bufs = (rows0, rows1)
    gsems = (gsem0, gsem1)
    osems = (osem0, osem1)

    gathers = [None] * _NCHUNK
    outs = [None] * _NCHUNK

    for i in range(min(_NBUF, _NCHUNK)):
        gathers[i] = pltpu.async_copy(
            table_hbm.at[idx_v.at[i]], bufs[i % _NBUF], gsems[i % _NBUF])

    for i in range(_NCHUNK):
        gathers[i].wait()
        outs[i] = pltpu.async_copy(
            bufs[i % _NBUF], out_hbm.at[pl.ds(base + i * _CHUNK, _CHUNK)],
            osems[i % _NBUF])
        j = i + _NBUF
        if j < _NCHUNK:
            outs[j - _NBUF].wait()
            gathers[j] = pltpu.async_copy(
                table_hbm.at[idx_v.at[j]], bufs[j % _NBUF], gsems[j % _NBUF])

    for i in range(max(0, _NCHUNK - _NBUF), _NCHUNK):
        outs[i].wait()


def kernel(X, weight):
    idx = jnp.asarray(X, jnp.int32).reshape(_NW, _NCHUNK, _CHUNK)
    out = _emb_lookup(idx, weight)
    return out.reshape(X.shape[0], X.shape[1], _EMB)


# speedup vs baseline: 1.2165x; 1.2165x over previous
import functools

import jax
import jax.numpy as jnp
from jax import lax
from jax.experimental import pallas as pl
from jax.experimental.pallas import tpu as pltpu
from jax.experimental.pallas import tpu_sc as plsc

_EMB = 64
_N = 4096 * 26
_NC = 2
_NS = 16
_NW = _NC * _NS
_BPW = _N // _NW
_NBUF = 2
_CHUNK = 832
_NCHUNK = _BPW // _CHUNK

_mesh = plsc.VectorSubcoreMesh(core_axis_name="c", subcore_axis_name="s")


@functools.partial(
    pl.kernel,
    mesh=_mesh,
    out_type=jax.ShapeDtypeStruct((_N, _EMB), jnp.float32),
    compiler_params=pltpu.CompilerParams(use_tc_tiling_on_sc=False),
    scratch_types=[
        pltpu.VMEM((_BPW,), jnp.int32),
        pltpu.VMEM((_CHUNK, _EMB), jnp.float32),
        pltpu.VMEM((_CHUNK, _EMB), jnp.float32),
        pltpu.SemaphoreType.DMA,
        pltpu.SemaphoreType.DMA,
        pltpu.SemaphoreType.DMA,
        pltpu.SemaphoreType.DMA,
    ],
)
def _emb_lookup(idx_hbm, table_hbm, out_hbm, idx_v, rows0, rows1,
                gsem0, gsem1, osem0, osem1):
    wid = lax.axis_index("s") * _NC + lax.axis_index("c")
    base = wid * _BPW

    pltpu.sync_copy(idx_hbm.at[pl.ds(base, _BPW)], idx_v)

    bufs = (rows0, rows1)
    gsems = (gsem0, gsem1)
    osems = (osem0, osem1)

    gathers = [None] * _NCHUNK
    outs = [None] * _NCHUNK

    for i in range(min(_NBUF, _NCHUNK)):
        gathers[i] = pltpu.async_copy(
            table_hbm.at[idx_v.at[pl.ds(i * _CHUNK, _CHUNK)]],
            bufs[i % _NBUF], gsems[i % _NBUF])

    for i in range(_NCHUNK):
        gathers[i].wait()
        outs[i] = pltpu.async_copy(
            bufs[i % _NBUF], out_hbm.at[pl.ds(base + i * _CHUNK, _CHUNK)],
            osems[i % _NBUF])
        j = i + _NBUF
        if j < _NCHUNK:
            outs[j - _NBUF].wait()
            gathers[j] = pltpu.async_copy(
                table_hbm.at[idx_v.at[pl.ds(j * _CHUNK, _CHUNK)]],
                bufs[j % _NBUF], gsems[j % _NBUF])

    for i in range(max(0, _NCHUNK - _NBUF), _NCHUNK):
        outs[i].wait()


def kernel(X, weight):
    idx = jnp.asarray(X, jnp.int32).reshape(_N)
    out = _emb_lookup(idx, weight)
    return out.reshape(X.shape[0], X.shape[1], _EMB)


# speedup vs baseline: 1.2233x; 1.0056x over previous
import functools

import jax
import jax.numpy as jnp
from jax import lax
from jax.experimental import pallas as pl
from jax.experimental.pallas import tpu as pltpu
from jax.experimental.pallas import tpu_sc as plsc

_EMB = 64
_N = 4096 * 26
_NC = 2
_NS = 16
_NW = _NC * _NS
_BPW = _N // _NW
_NBUF = 4
_CHUNK = 416
_NCHUNK = _BPW // _CHUNK

_mesh = plsc.VectorSubcoreMesh(core_axis_name="c", subcore_axis_name="s")


@functools.partial(
    pl.kernel,
    mesh=_mesh,
    out_type=jax.ShapeDtypeStruct((_N, _EMB), jnp.float32),
    compiler_params=pltpu.CompilerParams(use_tc_tiling_on_sc=False),
    scratch_types=[
        pltpu.VMEM((_BPW,), jnp.int32),
        *([pltpu.VMEM((_CHUNK, _EMB), jnp.float32)] * _NBUF),
        *([pltpu.SemaphoreType.DMA] * (2 * _NBUF)),
    ],
)
def _emb_lookup(idx_hbm, table_hbm, out_hbm, idx_v, *bufs_and_sems):
    bufs = bufs_and_sems[:_NBUF]
    gsems = bufs_and_sems[_NBUF:2 * _NBUF]
    osems = bufs_and_sems[2 * _NBUF:3 * _NBUF]
    wid = lax.axis_index("s") * _NC + lax.axis_index("c")
    base = wid * _BPW

    pltpu.sync_copy(idx_hbm.at[pl.ds(base, _BPW)], idx_v)

    gathers = [None] * _NCHUNK
    outs = [None] * _NCHUNK

    for i in range(min(_NBUF, _NCHUNK)):
        gathers[i] = pltpu.async_copy(
            table_hbm.at[idx_v.at[pl.ds(i * _CHUNK, _CHUNK)]],
            bufs[i % _NBUF], gsems[i % _NBUF])

    for i in range(_NCHUNK):
        gathers[i].wait()
        outs[i] = pltpu.async_copy(
            bufs[i % _NBUF], out_hbm.at[pl.ds(base + i * _CHUNK, _CHUNK)],
            osems[i % _NBUF])
        j = i + _NBUF
        if j < _NCHUNK:
            outs[j - _NBUF].wait()
            gathers[j] = pltpu.async_copy(
                table_hbm.at[idx_v.at[pl.ds(j * _CHUNK, _CHUNK)]],
                bufs[j % _NBUF], gsems[j % _NBUF])

    for i in range(max(0, _NCHUNK - _NBUF), _NCHUNK):
        outs[i].wait()


def kernel(X, weight):
    idx = jnp.asarray(X, jnp.int32).reshape(_N)
    out = _emb_lookup(idx, weight)
    return out.reshape(X.shape[0], X.shape[1], _EMB)
